# baseline (speedup 1.0000x reference)
import functools

import jax
import jax.numpy as jnp
from jax import lax
from jax.experimental import pallas as pl
from jax.experimental.pallas import tpu as pltpu
from jax.experimental.pallas import tpu_sc as plsc

N_SVC, N_INST, N_NODE = 10000, 80000, 10000
D = 128
E_SC, E_IN, E_NI = 100000, 256000, 256000

NC, NS, L = 2, 16, 16
CHUNK = 128


def _pad_edges(a, epad):
  return jnp.concatenate([a.astype(jnp.int32),
                          jnp.full((epad - a.shape[0],), -1, jnp.int32)])


def _round_up(x, m):
  return (x + m - 1) // m * m


_HBIG, _HSMALL = 81920, 10240
_HHALF = _HBIG // 2


_SBH = 2000
_ZN = 16384


def _emit_hist(edge_refs, etiles, bins_list, outs, hist_v, chunk_v, zh,
               stage, sid):
  ones16 = jnp.ones((16,), jnp.float32)

  for eref, etile, bins, out in zip(edge_refs, etiles, bins_list, outs):
    for z in range(bins // _ZN):
      pltpu.sync_copy(zh, hist_v.at[pl.ds(z * _ZN, _ZN)])
    if bins % _ZN:
      pltpu.sync_copy(zh.at[pl.ds(0, bins % _ZN)],
                      hist_v.at[pl.ds((bins // _ZN) * _ZN, bins % _ZN)])
    sbs = [_SBH] * (etile // _SBH)
    if etile % _SBH:
      sbs.append(etile % _SBH)
    base0 = sid * etile
    off = 0
    for sb in sbs:
      pltpu.sync_copy(eref.at[pl.ds(base0 + off, sb)],
                      chunk_v.at[pl.ds(0, sb)])
      def scan(j, _):
        v = chunk_v[pl.ds(j * 16, 16)]
        plsc.addupdate_scatter(hist_v, [v], ones16, mask=v >= 0)
        return _
      lax.fori_loop(0, sb // 16, scan, None)
      off += sb

    for lo_bin in range(0, bins, _HHALF):
      nb = min(_HHALF, bins - lo_bin)
      pltpu.sync_copy(hist_v.at[pl.ds(lo_bin, nb)],
                      stage.at[pl.ds(sid * nb, nb)])
      plsc.subcore_barrier()
      w = nb // NS
      for j in range(NS):
        pltpu.sync_copy(stage.at[pl.ds(j * nb + sid * w, w)],
                        hist_v.at[pl.ds(j * w, w)])
      def red2(i, _):
        acc = hist_v[pl.ds(i * 16, 16)]
        for j in range(1, NS):
          acc = acc + hist_v[pl.ds(j * w + i * 16, 16)]
        hist_v[pl.ds(i * 16, 16)] = acc
        return _
      lax.fori_loop(0, w // 16, red2, None)
      pltpu.sync_copy(hist_v.at[pl.ds(0, w)],
                      out.at[pl.ds(lo_bin + sid * w, w)])
      plsc.subcore_barrier()


def _hist_body(e0, e1, e2, zh, o0, o1, o2, hist_v, chunk_v, stage,
               *, big_first):
  cid = lax.axis_index("c")
  sid = lax.axis_index("s")

  @pl.when(cid == 0)
  def _():
    _emit_hist((e0,), (e0.shape[0] // NS,), (_HBIG,), (o0,),
               hist_v, chunk_v, zh, stage, sid)

  @pl.when(cid == 1)
  def _():
    _emit_hist((e1, e2), (e1.shape[0] // NS, e2.shape[0] // NS),
               (_HSMALL, _HSMALL), (o1, o2),
               hist_v, chunk_v, zh, stage, sid)


def _hist_call(big, small1, small2):
  f = pl.kernel(
      functools.partial(_hist_body, big_first=True),
      out_type=(
          jax.ShapeDtypeStruct((_HBIG,), jnp.float32),
          jax.ShapeDtypeStruct((_HSMALL,), jnp.float32),
          jax.ShapeDtypeStruct((_HSMALL,), jnp.float32),
      ),
      mesh=plsc.VectorSubcoreMesh(core_axis_name="c", subcore_axis_name="s"),
      compiler_params=pltpu.CompilerParams(use_tc_tiling_on_sc=False,
                                           needs_layout_passes=False),
      scratch_types=[
          pltpu.VMEM((_HBIG,), jnp.float32),
          pltpu.VMEM((_SBH,), jnp.int32),
          pltpu.VMEM_SHARED((NS * _HHALF,), jnp.float32),
      ],
  )
  return f(big, small1, small2, jnp.zeros((_ZN,), jnp.float32))


def _matmul_body(x_ref, deg_ref, w_ref, o_ref):
  scale = lax.rsqrt(jnp.maximum(deg_ref[...], 1.0))
  o_ref[...] = jnp.dot(x_ref[...] * scale, w_ref[...],
                       preferred_element_type=jnp.float32
                       ).astype(jnp.bfloat16)


def _matmul_call(x, deg2d, w, blk):
  n = x.shape[0]
  return pl.pallas_call(
      _matmul_body,
      grid=(n // blk,),
      in_specs=[
          pl.BlockSpec((blk, D), lambda i: (i, 0)),
          pl.BlockSpec((blk, 1), lambda i: (i, 0)),
          pl.BlockSpec((D, D), lambda i: (0, 0)),
      ],
      out_specs=pl.BlockSpec((blk, D), lambda i: (i, 0)),
      out_shape=jax.ShapeDtypeStruct((n, D), jnp.bfloat16),
  )(x, deg2d, w)


NBUF = 4


def _edge_body(h, src, dst, zhbm, out, zb2, srcv, dstv, gidx, sidx, rows_v,
               agg_sp, sem_i, sem_j, sem_g, sem_s, *, parts, rows,
               split_edges=False):
  cid = lax.axis_index("c")
  sid = lax.axis_index("s")
  nworkers = NC * NS if split_edges else NS
  etile = src.shape[0] // nworkers
  ktot = etile // CHUNK
  kmain = (ktot // NBUF) * NBUF
  tail = ktot - kmain
  rpt = rows // NS
  half = cid * (parts * rows)
  pltpu.sync_copy(zhbm, zb2)

  wid = (cid * NS + sid) if split_edges else sid

  def base(k):
    return wid * etile + k * CHUNK

  def issue_idx(b, k):
    k = jnp.minimum(k, ktot - 1)
    pltpu.async_copy(src.at[pl.ds(base(k), CHUNK)], srcv.at[b], sem_i.at[b])
    pltpu.async_copy(dst.at[pl.ds(base(k), CHUNK)], dstv.at[b], sem_j.at[b])

  def wait_idx(b):
    pltpu.make_async_copy(src.at[pl.ds(0, CHUNK)], srcv.at[b],
                          sem_i.at[b]).wait()
    pltpu.make_async_copy(dst.at[pl.ds(0, CHUNK)], dstv.at[b],
                          sem_j.at[b]).wait()

  def remap(b, lo):
    for j in range(8):
      d16 = dstv[b, pl.ds(j * 16, 16)]
      s16 = srcv[b, pl.ds(j * 16, 16)]
      m = (d16 >= lo) & (d16 < lo + rows)
      gidx[b, pl.ds(j * 16, 16)] = jnp.where(m, s16, -1)
      sidx[b, pl.ds(j * 16, 16)] = jnp.where(m, d16 - lo, -1)

  def issue_gather(b):
    pltpu.async_copy(h.at[plsc.Indices(gidx.at[b], ignored_value=-1)],
                     rows_v.at[b], sem_g.at[b])

  def wait_gather(b):
    pltpu.make_async_copy(h.at[plsc.Indices(gidx.at[b], ignored_value=-1)],
                          rows_v.at[b], sem_g.at[b]).wait()

  def issue_scatter(b):
    pltpu.async_copy(rows_v.at[b],
                     agg_sp.at[plsc.Indices(sidx.at[b], ignored_value=-1)],
                     sem_s.at[b], add=True)

  def wait_scatter(b):
    pltpu.make_async_copy(rows_v.at[b],
                          agg_sp.at[plsc.Indices(sidx.at[b],
                                                 ignored_value=-1)],
                          sem_s.at[b]).wait()

  def cell(k, b, lo, s1, do_issue, s3):
    if s1:
      wait_scatter(b)
    wait_idx(b)
    remap(b, lo)
    issue_gather(b)
    if do_issue:
      issue_idx(b, k + NBUF)
    if s3:
      b2 = (b - 2) % NBUF
      wait_gather(b2)
      issue_scatter(b2)

  for p in range(parts):
    lo = 0 if split_edges else half + p * rows
    nfull, rem = rpt // 64, rpt % 64
    def zcp(j, _):
      pltpu.sync_copy(zb2, agg_sp.at[pl.ds(sid * rpt + j * 64, 64)])
      return _
    lax.fori_loop(0, nfull, zcp, None)
    if rem:
      pltpu.sync_copy(zb2.at[pl.ds(0, rem)],
                      agg_sp.at[pl.ds(sid * rpt + nfull * 64, rem)])
    plsc.subcore_barrier()

    for b in range(NBUF):
      issue_idx(b, b)
    for k in range(NBUF):
      cell(k, k, lo, s1=False, do_issue=True, s3=(k >= 2))

    def group(g, _):
      for b in range(NBUF):
        cell(g * NBUF + b, b, lo, s1=True, do_issue=True, s3=True)
      return _
    lax.fori_loop(1, kmain // NBUF, group, None)

    for k in range(kmain, ktot):
      cell(k, k % NBUF, lo, s1=True, do_issue=True, s3=True)
    for k in range(ktot - 2, ktot):
      b2 = k % NBUF
      wait_gather(b2)
      issue_scatter(b2)
    for k in range(ktot - NBUF, ktot):
      wait_scatter(k % NBUF)
    for k in range(ktot, ktot + NBUF):
      wait_idx(k % NBUF)

    plsc.subcore_barrier()
    dump_off = (cid * rows + sid * rpt) if split_edges else lo + sid * rpt
    pltpu.sync_copy(agg_sp.at[pl.ds(sid * rpt, rpt)],
                    out.at[pl.ds(dump_off, rpt)])
    plsc.subcore_barrier()


def _edge_call(h, src_pad, dst_pad, parts, rows, split_edges=False):
  f = pl.kernel(
      functools.partial(_edge_body, parts=parts, rows=rows,
                        split_edges=split_edges),
      out_type=jax.ShapeDtypeStruct((NC * parts * rows, D), jnp.bfloat16),
      mesh=plsc.VectorSubcoreMesh(core_axis_name="c", subcore_axis_name="s"),
      compiler_params=pltpu.CompilerParams(use_tc_tiling_on_sc=False,
                                           needs_layout_passes=False),
      scratch_types=[
          pltpu.VMEM((64, D), jnp.bfloat16),
          pltpu.VMEM((NBUF, CHUNK), jnp.int32),
          pltpu.VMEM((NBUF, CHUNK), jnp.int32),
          pltpu.VMEM((NBUF, CHUNK), jnp.int32),
          pltpu.VMEM((NBUF, CHUNK), jnp.int32),
          pltpu.VMEM((NBUF, CHUNK, D), jnp.bfloat16),
          pltpu.VMEM_SHARED((rows, D), jnp.bfloat16),
          pltpu.SemaphoreType.DMA((NBUF,)),
          pltpu.SemaphoreType.DMA((NBUF,)),
          pltpu.SemaphoreType.DMA((NBUF,)),
          pltpu.SemaphoreType.DMA((NBUF,)),
      ],
  )
  return f(h, src_pad, dst_pad, jnp.zeros((64, D), jnp.bfloat16))


def _finalize_body(agg_ref, deg_ref, b_ref, o_ref):
  t = (agg_ref[...].astype(jnp.float32)
       * lax.rsqrt(jnp.maximum(deg_ref[...], 1.0)) + b_ref[...])
  o_ref[...] = jnp.where(t >= 0, t, 0.01 * t)


def _finalize_first_body(agg_ref, deg_ref, b_ref, o_ref):
  _finalize_body(agg_ref, deg_ref, b_ref, o_ref)


def _finalize_alias_body(buf_ref, agg_ref, deg_ref, b_ref, o_ref):
  _finalize_body(agg_ref, deg_ref, b_ref, o_ref)


def _finalize2_body(agg_ref, agg2_ref, deg_ref, b_ref, o_ref):
  a = agg_ref[...].astype(jnp.float32) + agg2_ref[...].astype(jnp.float32)
  t = a * lax.rsqrt(jnp.maximum(deg_ref[...], 1.0)) + b_ref[...]
  o_ref[...] = jnp.where(t >= 0, t, 0.01 * t)


def _finalize2_alias_body(buf_ref, agg_ref, agg2_ref, deg_ref, b_ref, o_ref):
  _finalize2_body(agg_ref, agg2_ref, deg_ref, b_ref, o_ref)


_NTOT = N_NODE + N_INST + N_SVC
_FBLK = 2000


def _finalize_seg(buf, aggs, deg2d, b, n, seg0):
  grid = (n // _FBLK,)
  sb = seg0 // _FBLK
  nagg = len(aggs)
  specs = ([pl.BlockSpec((_FBLK, D), lambda i: (i, 0))] * nagg + [
      pl.BlockSpec((_FBLK, 1), lambda i: (i, 0)),
      pl.BlockSpec((1, D), lambda i: (0, 0)),
  ])
  out_spec = pl.BlockSpec((_FBLK, D), lambda i: (i + sb, 0))
  out_shape = jax.ShapeDtypeStruct((_NTOT, D), jnp.float32)
  body1 = _finalize_first_body if nagg == 1 else _finalize2_body
  body2 = _finalize_alias_body if nagg == 1 else _finalize2_alias_body
  if buf is None:
    return pl.pallas_call(
        body1, grid=grid, in_specs=specs,
        out_specs=out_spec, out_shape=out_shape,
    )(*aggs, deg2d, b.reshape(1, D))
  return pl.pallas_call(
      body2, grid=grid,
      in_specs=[pl.BlockSpec(memory_space=pl.ANY)] + specs,
      out_specs=out_spec, out_shape=out_shape,
      input_output_aliases={0: 0},
  )(buf, *aggs, deg2d, b.reshape(1, D))


def kernel(x_svc, x_inst, x_node, ei_svc_src, ei_svc_dst, ei_in_src,
           ei_in_dst, ei_ni_src, ei_ni_dst, W_svc, b_svc, W_inst, b_inst,
           W_node, b_node):
  esc_pad = _round_up(E_SC, NC * NS * CHUNK)
  ein_pad = _round_up(E_IN, NC * NS * CHUNK)
  eni_pad = _round_up(E_NI, NS * CHUNK)
  svc_src = _pad_edges(ei_svc_src, esc_pad)
  svc_dst = _pad_edges(ei_svc_dst, esc_pad)
  in_src = _pad_edges(ei_in_src, ein_pad)
  in_dst = _pad_edges(ei_in_dst, ein_pad)
  ni_src = _pad_edges(ei_ni_src, eni_pad)
  ni_dst = _pad_edges(ei_ni_dst, eni_pad)

  h_in_src, h_ni_src, h_svc_src = _hist_call(in_src, ni_src, svc_src)
  h_ni_dst, h_in_dst, h_svc_dst = _hist_call(ni_dst, in_dst, svc_dst)

  def deg2d(hist, n):
    return hist.reshape(-1, 1)[:n]

  h_svc = _matmul_call(x_svc, deg2d(h_svc_src, N_SVC), W_svc, 2000)
  h_inst = _matmul_call(x_inst, deg2d(h_in_src, N_INST), W_inst, 2000)
  h_node = _matmul_call(x_node, deg2d(h_ni_src, N_NODE), W_node, 2000)

  agg_svc = _edge_call(h_svc, svc_src, svc_dst, 1, 10240,
                       split_edges=True)
  agg_node = _edge_call(h_inst, in_src, in_dst, 1, 10240,
                        split_edges=True)
  agg_inst = _edge_call(h_node, ni_src, ni_dst, 3, 13952)

  buf = _finalize_seg(None, (agg_node[:10240], agg_node[10240:]),
                      deg2d(h_in_dst, N_NODE), b_inst, N_NODE, 0)
  buf = _finalize_seg(buf, (agg_inst,), deg2d(h_ni_dst, N_INST), b_node,
                      N_INST, N_NODE)
  buf = _finalize_seg(buf, (agg_svc[:10240], agg_svc[10240:]),
                      deg2d(h_svc_dst, N_SVC), b_svc,
                      N_SVC, N_NODE + N_INST)
  return buf


# speedup vs baseline: 1.0664x; 1.0664x over previous
import functools

import jax
import jax.numpy as jnp
from jax import lax
from jax.experimental import pallas as pl
from jax.experimental.pallas import tpu as pltpu
from jax.experimental.pallas import tpu_sc as plsc

N_SVC, N_INST, N_NODE = 10000, 80000, 10000
D = 128
E_SC, E_IN, E_NI = 100000, 256000, 256000

NC, NS, L = 2, 16, 16
CHUNK = 128


def _pad_edges(a, epad):
  return jnp.concatenate([a.astype(jnp.int32),
                          jnp.full((epad - a.shape[0],), -1, jnp.int32)])


def _round_up(x, m):
  return (x + m - 1) // m * m


_HBIG, _HSMALL = 81920, 10240
_HHALF = _HBIG // 2


_SBH = 2000
_ZN = 16384


def _emit_hist(edge_refs, etiles, bins_list, outs, hist_v, chunk_v, zh,
               stage, sid):
  ones16 = jnp.ones((16,), jnp.float32)

  for eref, etile, bins, out in zip(edge_refs, etiles, bins_list, outs):
    for z in range(bins // _ZN):
      pltpu.sync_copy(zh, hist_v.at[pl.ds(z * _ZN, _ZN)])
    if bins % _ZN:
      pltpu.sync_copy(zh.at[pl.ds(0, bins % _ZN)],
                      hist_v.at[pl.ds((bins // _ZN) * _ZN, bins % _ZN)])
    sbs = [_SBH] * (etile // _SBH)
    if etile % _SBH:
      sbs.append(etile % _SBH)
    base0 = sid * etile
    off = 0
    for sb in sbs:
      pltpu.sync_copy(eref.at[pl.ds(base0 + off, sb)],
                      chunk_v.at[pl.ds(0, sb)])
      def scan(j, _):
        v = chunk_v[pl.ds(j * 16, 16)]
        plsc.addupdate_scatter(hist_v, [v], ones16, mask=v >= 0)
        return _
      lax.fori_loop(0, sb // 16, scan, None)
      off += sb

    for lo_bin in range(0, bins, _HHALF):
      nb = min(_HHALF, bins - lo_bin)
      pltpu.sync_copy(hist_v.at[pl.ds(lo_bin, nb)],
                      stage.at[pl.ds(sid * nb, nb)])
      plsc.subcore_barrier()
      w = nb // NS
      for j in range(NS):
        pltpu.sync_copy(stage.at[pl.ds(j * nb + sid * w, w)],
                        hist_v.at[pl.ds(j * w, w)])
      def red2(i, _):
        acc = hist_v[pl.ds(i * 16, 16)]
        for j in range(1, NS):
          acc = acc + hist_v[pl.ds(j * w + i * 16, 16)]
        hist_v[pl.ds(i * 16, 16)] = acc
        return _
      lax.fori_loop(0, w // 16, red2, None)
      pltpu.sync_copy(hist_v.at[pl.ds(0, w)],
                      out.at[pl.ds(lo_bin + sid * w, w)])
      plsc.subcore_barrier()


def _hist_body(e0, e1, e2, zh, o0, o1, o2, hist_v, chunk_v, stage,
               *, big_first):
  cid = lax.axis_index("c")
  sid = lax.axis_index("s")

  @pl.when(cid == 0)
  def _():
    _emit_hist((e0,), (e0.shape[0] // NS,), (_HBIG,), (o0,),
               hist_v, chunk_v, zh, stage, sid)

  @pl.when(cid == 1)
  def _():
    _emit_hist((e1, e2), (e1.shape[0] // NS, e2.shape[0] // NS),
               (_HSMALL, _HSMALL), (o1, o2),
               hist_v, chunk_v, zh, stage, sid)


def _hist_call(big, small1, small2):
  f = pl.kernel(
      functools.partial(_hist_body, big_first=True),
      out_type=(
          jax.ShapeDtypeStruct((_HBIG,), jnp.float32),
          jax.ShapeDtypeStruct((_HSMALL,), jnp.float32),
          jax.ShapeDtypeStruct((_HSMALL,), jnp.float32),
      ),
      mesh=plsc.VectorSubcoreMesh(core_axis_name="c", subcore_axis_name="s"),
      compiler_params=pltpu.CompilerParams(use_tc_tiling_on_sc=False,
                                           needs_layout_passes=False),
      scratch_types=[
          pltpu.VMEM((_HBIG,), jnp.float32),
          pltpu.VMEM((_SBH,), jnp.int32),
          pltpu.VMEM_SHARED((NS * _HHALF,), jnp.float32),
      ],
  )
  return f(big, small1, small2, jnp.zeros((_ZN,), jnp.float32))


def _matmul_body(x_ref, deg_ref, w_ref, o_ref):
  scale = lax.rsqrt(jnp.maximum(deg_ref[...], 1.0))
  o_ref[...] = jnp.dot(x_ref[...] * scale, w_ref[...],
                       preferred_element_type=jnp.float32
                       ).astype(jnp.bfloat16)


def _matmul_call(x, deg2d, w, blk):
  n = x.shape[0]
  return pl.pallas_call(
      _matmul_body,
      grid=(n // blk,),
      in_specs=[
          pl.BlockSpec((blk, D), lambda i: (i, 0)),
          pl.BlockSpec((blk, 1), lambda i: (i, 0)),
          pl.BlockSpec((D, D), lambda i: (0, 0)),
      ],
      out_specs=pl.BlockSpec((blk, D), lambda i: (i, 0)),
      out_shape=jax.ShapeDtypeStruct((n, D), jnp.bfloat16),
  )(x, deg2d, w)


NBUF = 4


def _edge_body(h, src, dst, zhbm, out, zb2, srcv, dstv, gidx, sidx, rows_v,
               agg_sp, sem_i, sem_j, sem_g, sem_s, *, parts, rows):
  cid = lax.axis_index("c")
  sid = lax.axis_index("s")
  etile = src.shape[0] // NS
  ktot = etile // CHUNK
  kmain = (ktot // NBUF) * NBUF
  tail = ktot - kmain
  rpt = rows // NS
  half = cid * (parts * rows)
  pltpu.sync_copy(zhbm, zb2)

  def base(k):
    return sid * etile + k * CHUNK

  def issue_idx(b, k):
    k = jnp.minimum(k, ktot - 1)
    pltpu.async_copy(src.at[pl.ds(base(k), CHUNK)], srcv.at[b], sem_i.at[b])
    pltpu.async_copy(dst.at[pl.ds(base(k), CHUNK)], dstv.at[b], sem_j.at[b])

  def wait_idx(b):
    pltpu.make_async_copy(src.at[pl.ds(0, CHUNK)], srcv.at[b],
                          sem_i.at[b]).wait()
    pltpu.make_async_copy(dst.at[pl.ds(0, CHUNK)], dstv.at[b],
                          sem_j.at[b]).wait()

  def remap(b, lo):
    for j in range(8):
      d16 = dstv[b, pl.ds(j * 16, 16)]
      s16 = srcv[b, pl.ds(j * 16, 16)]
      m = (d16 >= lo) & (d16 < lo + rows)
      gidx[b, pl.ds(j * 16, 16)] = jnp.where(m, s16, -1)
      sidx[b, pl.ds(j * 16, 16)] = jnp.where(m, d16 - lo, -1)

  def issue_gather(b):
    pltpu.async_copy(h.at[plsc.Indices(gidx.at[b], ignored_value=-1)],
                     rows_v.at[b], sem_g.at[b])

  def wait_gather(b):
    pltpu.make_async_copy(h.at[plsc.Indices(gidx.at[b], ignored_value=-1)],
                          rows_v.at[b], sem_g.at[b]).wait()

  def issue_scatter(b):
    pltpu.async_copy(rows_v.at[b],
                     agg_sp.at[plsc.Indices(sidx.at[b], ignored_value=-1)],
                     sem_s.at[b], add=True)

  def wait_scatter(b):
    pltpu.make_async_copy(rows_v.at[b],
                          agg_sp.at[plsc.Indices(sidx.at[b],
                                                 ignored_value=-1)],
                          sem_s.at[b]).wait()

  def cell(k, b, lo, s1, do_issue, s3):
    if s1:
      wait_scatter(b)
    wait_idx(b)
    remap(b, lo)
    issue_gather(b)
    if do_issue:
      issue_idx(b, k + NBUF)
    if s3:
      b2 = (b - 2) % NBUF
      wait_gather(b2)
      issue_scatter(b2)

  for p in range(parts):
    lo = half + p * rows
    nfull, rem = rpt // 64, rpt % 64
    def zcp(j, _):
      pltpu.sync_copy(zb2, agg_sp.at[pl.ds(sid * rpt + j * 64, 64)])
      return _
    lax.fori_loop(0, nfull, zcp, None)
    if rem:
      pltpu.sync_copy(zb2.at[pl.ds(0, rem)],
                      agg_sp.at[pl.ds(sid * rpt + nfull * 64, rem)])
    plsc.subcore_barrier()

    for b in range(NBUF):
      issue_idx(b, b)
    for k in range(NBUF):
      cell(k, k, lo, s1=False, do_issue=True, s3=(k >= 2))

    def group(g, _):
      for b in range(NBUF):
        cell(g * NBUF + b, b, lo, s1=True, do_issue=True, s3=True)
      return _
    lax.fori_loop(1, kmain // NBUF, group, None)

    for k in range(kmain, ktot):
      cell(k, k % NBUF, lo, s1=True, do_issue=True, s3=True)
    for k in range(ktot - 2, ktot):
      b2 = k % NBUF
      wait_gather(b2)
      issue_scatter(b2)
    for k in range(ktot - NBUF, ktot):
      wait_scatter(k % NBUF)
    for k in range(ktot, ktot + NBUF):
      wait_idx(k % NBUF)

    plsc.subcore_barrier()
    pltpu.sync_copy(agg_sp.at[pl.ds(sid * rpt, rpt)],
                    out.at[pl.ds(lo + sid * rpt, rpt)])
    plsc.subcore_barrier()


def _edge_call(h, src_pad, dst_pad, parts, rows):
  f = pl.kernel(
      functools.partial(_edge_body, parts=parts, rows=rows),
      out_type=jax.ShapeDtypeStruct((NC * parts * rows, D), jnp.bfloat16),
      mesh=plsc.VectorSubcoreMesh(core_axis_name="c", subcore_axis_name="s"),
      compiler_params=pltpu.CompilerParams(use_tc_tiling_on_sc=False,
                                           needs_layout_passes=False),
      scratch_types=[
          pltpu.VMEM((64, D), jnp.bfloat16),
          pltpu.VMEM((NBUF, CHUNK), jnp.int32),
          pltpu.VMEM((NBUF, CHUNK), jnp.int32),
          pltpu.VMEM((NBUF, CHUNK), jnp.int32),
          pltpu.VMEM((NBUF, CHUNK), jnp.int32),
          pltpu.VMEM((NBUF, CHUNK, D), jnp.bfloat16),
          pltpu.VMEM_SHARED((rows, D), jnp.bfloat16),
          pltpu.SemaphoreType.DMA((NBUF,)),
          pltpu.SemaphoreType.DMA((NBUF,)),
          pltpu.SemaphoreType.DMA((NBUF,)),
          pltpu.SemaphoreType.DMA((NBUF,)),
      ],
  )
  return f(h, src_pad, dst_pad, jnp.zeros((64, D), jnp.bfloat16))


def _finalize_body(agg_ref, deg_ref, b_ref, o_ref):
  t = (agg_ref[...].astype(jnp.float32)
       * lax.rsqrt(jnp.maximum(deg_ref[...], 1.0)) + b_ref[...])
  o_ref[...] = jnp.where(t >= 0, t, 0.01 * t)


def _finalize_first_body(agg_ref, deg_ref, b_ref, o_ref):
  _finalize_body(agg_ref, deg_ref, b_ref, o_ref)


def _finalize_alias_body(buf_ref, agg_ref, deg_ref, b_ref, o_ref):
  _finalize_body(agg_ref, deg_ref, b_ref, o_ref)


_NTOT = N_NODE + N_INST + N_SVC
_FBLK = 2000


def _finalize_seg(buf, agg_pad, deg2d, b, n, seg0):
  grid = (n // _FBLK,)
  sb = seg0 // _FBLK
  specs = [
      pl.BlockSpec((_FBLK, D), lambda i: (i, 0)),
      pl.BlockSpec((_FBLK, 1), lambda i: (i, 0)),
      pl.BlockSpec((1, D), lambda i: (0, 0)),
  ]
  out_spec = pl.BlockSpec((_FBLK, D), lambda i: (i + sb, 0))
  out_shape = jax.ShapeDtypeStruct((_NTOT, D), jnp.float32)
  if buf is None:
    return pl.pallas_call(
        _finalize_first_body, grid=grid, in_specs=specs,
        out_specs=out_spec, out_shape=out_shape,
    )(agg_pad, deg2d, b.reshape(1, D))
  return pl.pallas_call(
      _finalize_alias_body, grid=grid,
      in_specs=[pl.BlockSpec(memory_space=pl.ANY)] + specs,
      out_specs=out_spec, out_shape=out_shape,
      input_output_aliases={0: 0},
  )(buf, agg_pad, deg2d, b.reshape(1, D))


def kernel(x_svc, x_inst, x_node, ei_svc_src, ei_svc_dst, ei_in_src,
           ei_in_dst, ei_ni_src, ei_ni_dst, W_svc, b_svc, W_inst, b_inst,
           W_node, b_node):
  esc_pad = _round_up(E_SC, NS * CHUNK)
  ein_pad = _round_up(E_IN, NS * CHUNK)
  eni_pad = _round_up(E_NI, NS * CHUNK)
  svc_src = _pad_edges(ei_svc_src, esc_pad)
  svc_dst = _pad_edges(ei_svc_dst, esc_pad)
  in_src = _pad_edges(ei_in_src, ein_pad)
  in_dst = _pad_edges(ei_in_dst, ein_pad)
  ni_src = _pad_edges(ei_ni_src, eni_pad)
  ni_dst = _pad_edges(ei_ni_dst, eni_pad)

  h_in_src, h_ni_src, h_svc_src = _hist_call(in_src, ni_src, svc_src)
  h_ni_dst, h_in_dst, h_svc_dst = _hist_call(ni_dst, in_dst, svc_dst)

  def deg2d(hist, n):
    return hist.reshape(-1, 1)[:n]

  h_svc = _matmul_call(x_svc, deg2d(h_svc_src, N_SVC), W_svc, 2000)
  h_inst = _matmul_call(x_inst, deg2d(h_in_src, N_INST), W_inst, 2000)
  h_node = _matmul_call(x_node, deg2d(h_ni_src, N_NODE), W_node, 2000)

  agg_svc = _edge_call(h_svc, svc_src, svc_dst, 1, 5120)
  agg_node = _edge_call(h_inst, in_src, in_dst, 1, 5120)
  agg_inst = _edge_call(h_node, ni_src, ni_dst, 3, 13952)

  buf = _finalize_seg(None, agg_node, deg2d(h_in_dst, N_NODE), b_inst,
                      N_NODE, 0)
  buf = _finalize_seg(buf, agg_inst, deg2d(h_ni_dst, N_INST), b_node,
                      N_INST, N_NODE)
  buf = _finalize_seg(buf, agg_svc, deg2d(h_svc_dst, N_SVC), b_svc,
                      N_SVC, N_NODE + N_INST)
  return buf
